# baseline (speedup 1.0000x reference)
import functools

import jax
import jax.numpy as jnp
from jax import lax
from jax.experimental import pallas as pl
from jax.experimental.pallas import tpu as pltpu
from jax.experimental.pallas import tpu_sc as plsc

NUM_EMB = 1_000_000
DIM = 32
ROWS = 16384
COLS = 26
B_TOTAL = ROWS * COLS
NC = 2
NS = 16
NW = NC * NS
B_PER_W = B_TOTAL // NW
CHUNK = 128
N_CHUNKS = B_PER_W // CHUNK
NBUF = 4
N_GROUPS = N_CHUNKS // NBUF
PACK = 128 // DIM
L = 16
OUT_ROWS_PER_CHUNK = CHUNK * DIM // 128


def _emb_body(idx_hbm, table_hbm, out_hbm, idx_v, pidx_v, rows_v, stage_v,
              *sems):
    gsems = sems[:NBUF]
    wsems = sems[NBUF:]
    wid = lax.axis_index("s") * NC + lax.axis_index("c")
    out_base = wid * B_PER_W

    pltpu.sync_copy(idx_hbm.at[wid], idx_v)

    def gather(j, b):
        def prep(v, _):
            idx16 = idx_v[j, pl.ds(v * L, L)]
            pidx_v[b, pl.ds(v * L, L)] = lax.shift_right_logical(idx16, 2)
            return _

        lax.fori_loop(0, CHUNK // L, prep, 0, unroll=True)
        pltpu.make_async_copy(
            table_hbm.at[pidx_v.at[b]], rows_v.at[b], gsems[b]
        ).start()

    def gather_wait(j, b):
        pltpu.make_async_copy(
            table_hbm.at[pidx_v.at[b]], rows_v.at[b], gsems[b]
        ).wait()

    def extract(j, b):
        def grp(g, carry):
            rawv = idx_v[j, pl.ds(g * L, L)]
            offv = lax.shift_left(lax.bitwise_and(rawv, PACK - 1), 5)
            for l in range(L):
                k = g * L + l
                off = pl.multiple_of(offv[l], DIM)
                for h in range(DIM // L):
                    stage_v[b, k, pl.ds(h * L, L)] = (
                        rows_v[b, k, pl.ds(off + h * L, L)])
            return carry

        lax.fori_loop(0, CHUNK // L, grp, 0)

    def write(j, b):
        pltpu.make_async_copy(
            stage_v.at[b],
            out_hbm.at[pl.ds(out_base + j * CHUNK, CHUNK)],
            wsems[b],
        ).start()

    def write_wait(j, b):
        pltpu.make_async_copy(
            stage_v.at[b],
            out_hbm.at[pl.ds(out_base + j * CHUNK, CHUNK)],
            wsems[b],
        ).wait()

    for b in range(NBUF - 1):
        gather(b, b)

    def body(g, carry):
        j0 = g * NBUF
        for b in range(NBUF):
            j = j0 + b
            gather_wait(j, b)
            extract(j, b)
            write(j, b)
            bp = (b - 1) % NBUF
            jn = j + NBUF - 1

            if b == 0:
                @pl.when(j >= 1)
                def _():
                    write_wait(j - 1, bp)

                gather(jn, bp)
            else:
                @pl.when(jn < N_CHUNKS)
                def _():
                    write_wait(j - 1, bp)
                    gather(jn, bp)

        return carry

    lax.fori_loop(0, N_GROUPS, body, 0)

    for b in range(NBUF):
        write_wait(N_CHUNKS - NBUF + b, b)


def kernel(input, weight):
    idx = input.reshape(-1).astype(jnp.int32)
    idx3 = idx.reshape(NW, N_CHUNKS, CHUNK)

    mesh = plsc.VectorSubcoreMesh(core_axis_name="c", subcore_axis_name="s")
    run = pl.kernel(
        _emb_body,
        out_type=jax.ShapeDtypeStruct((B_TOTAL, DIM), jnp.float32),
        mesh=mesh,
        scratch_types=[
            pltpu.VMEM((N_CHUNKS, CHUNK), jnp.int32),
            pltpu.VMEM((NBUF, CHUNK), jnp.int32),
            pltpu.VMEM((NBUF, CHUNK, 128), jnp.float32),
            pltpu.VMEM((NBUF, CHUNK, DIM), jnp.float32),
        ]
        + [pltpu.SemaphoreType.DMA] * (2 * NBUF),
        compiler_params=pltpu.CompilerParams(use_tc_tiling_on_sc=False),
    )
    out = run(idx3, weight.reshape(NUM_EMB // PACK, 128))
    return out.reshape(ROWS, COLS, DIM)


# speedup vs baseline: 1.3787x; 1.3787x over previous
import functools

import jax
import jax.numpy as jnp
from jax import lax
from jax.experimental import pallas as pl
from jax.experimental.pallas import tpu as pltpu
from jax.experimental.pallas import tpu_sc as plsc

NUM_EMB = 1_000_000
DIM = 32
ROWS = 16384
COLS = 26
B_TOTAL = ROWS * COLS
NC = 2
NS = 16
NW = NC * NS
B_PER_W = B_TOTAL // NW
CHUNK = 128
N_CHUNKS = B_PER_W // CHUNK
NBUF = 8
N_GROUPS = N_CHUNKS // NBUF
OUT_SUBROWS = ROWS * 32 * 128 // DIM


def _emb_body(idx_hbm, qidx_hbm, table_hbm, out_hbm, idx_v, qidx_v, rows_v,
              *sems):
    gsems = sems[:NBUF]
    wsems = sems[NBUF:]
    wid = lax.axis_index("s") * NC + lax.axis_index("c")

    pltpu.sync_copy(idx_hbm.at[wid], idx_v)
    pltpu.sync_copy(qidx_hbm.at[wid], qidx_v)

    def gather(j, b):
        pltpu.make_async_copy(
            table_hbm.at[idx_v.at[j]], rows_v.at[b], gsems[b]
        ).start()

    def gather_wait(j, b):
        pltpu.make_async_copy(
            table_hbm.at[idx_v.at[j]], rows_v.at[b], gsems[b]
        ).wait()

    def write(j, b):
        pltpu.make_async_copy(
            rows_v.at[b], out_hbm.at[qidx_v.at[j]], wsems[b]
        ).start()

    def write_wait(j, b):
        pltpu.make_async_copy(
            rows_v.at[b], out_hbm.at[qidx_v.at[j]], wsems[b]
        ).wait()

    for b in range(NBUF - 1):
        gather(b, b)

    def body(g, carry):
        j0 = g * NBUF
        for b in range(NBUF):
            j = j0 + b
            gather_wait(j, b)
            write(j, b)
            bp = (b - 1) % NBUF
            jn = j + NBUF - 1

            if b == 0:
                @pl.when(j >= 1)
                def _():
                    write_wait(j - 1, bp)

                gather(jn, bp)
            else:
                @pl.when(jn < N_CHUNKS)
                def _():
                    write_wait(j - 1, bp)
                    gather(jn, bp)

        return carry

    lax.fori_loop(0, N_GROUPS, body, 0)

    for b in range(NBUF):
        write_wait(N_CHUNKS - NBUF + b, b)


def kernel(input, weight):
    idx = input.reshape(-1).astype(jnp.int32)
    idx3 = idx.reshape(NW, N_CHUNKS, CHUNK)
    t = jnp.arange(B_TOTAL, dtype=jnp.int32)
    q = (t // COLS) * 128 + (t % COLS) * (128 // DIM)
    qidx3 = q.reshape(NW, N_CHUNKS, CHUNK)

    mesh = plsc.VectorSubcoreMesh(core_axis_name="c", subcore_axis_name="s")
    run = pl.kernel(
        _emb_body,
        out_type=jax.ShapeDtypeStruct((OUT_SUBROWS, DIM), jnp.float32),
        mesh=mesh,
        scratch_types=[
            pltpu.VMEM((N_CHUNKS, CHUNK), jnp.int32),
            pltpu.VMEM((N_CHUNKS, CHUNK), jnp.int32),
            pltpu.VMEM((NBUF, CHUNK, DIM), jnp.float32),
        ]
        + [pltpu.SemaphoreType.DMA] * (2 * NBUF),
        compiler_params=pltpu.CompilerParams(use_tc_tiling_on_sc=False),
    )
    out = run(idx3, qidx3, weight)
    return out.reshape(ROWS, 32, 128)[:, :COLS, :DIM]
